# baseline (speedup 1.0000x reference)
import functools

import jax
import jax.numpy as jnp
from jax import lax
from jax.experimental import pallas as pl
from jax.experimental.pallas import tpu as pltpu
from jax.experimental.pallas import tpu_sc as plsc

B = 16384
D = 32
L = 16
NC = 2
NS = 16
NW = NC * NS
BPW = B // NW
CHUNK = 128
NCHUNK = BPW // CHUNK


def _body(user_ref, item_ref, ue_ref, ie_ref, ub_ref, ib_ref, out_ref,
          uidx, iidx, urows, irows, ubias, ibias, outv, sem):
    wid = lax.axis_index("s") * NC + lax.axis_index("c")

    pltpu.sync_copy(user_ref.at[wid], uidx)
    pltpu.sync_copy(item_ref.at[wid], iidx)

    copies = []
    for j in range(NCHUNK):
        sl = pl.ds(j * CHUNK, CHUNK)
        copies.append(pltpu.async_copy(ue_ref.at[uidx.at[j]], urows.at[sl], sem))
        copies.append(pltpu.async_copy(ie_ref.at[iidx.at[j]], irows.at[sl], sem))
        copies.append(pltpu.async_copy(ub_ref.at[uidx.at[j]], ubias.at[sl], sem))
        copies.append(pltpu.async_copy(ib_ref.at[iidx.at[j]], ibias.at[sl], sem))
    for c in copies:
        c.wait()

    def group(g, carry):
        base = g * L
        rows = lax.iota(jnp.int32, L) + base
        acc = ubias[pl.ds(base, L)] + ibias[pl.ds(base, L)]
        for d in range(D):
            dsplat = jnp.full((L,), d, jnp.int32)
            uv = plsc.load_gather(urows, [rows, dsplat])
            iv = plsc.load_gather(irows, [rows, dsplat])
            acc = acc + uv * iv
        outv[pl.ds(base, L)] = acc
        return carry

    lax.fori_loop(0, BPW // L, group, 0)
    pltpu.sync_copy(outv, out_ref.at[pl.ds(wid * BPW, BPW)])


def kernel(user, item, user_emb, item_emb, user_bias, item_bias):
    user_r = user.astype(jnp.int32).reshape(NW, NCHUNK, CHUNK)
    item_r = item.astype(jnp.int32).reshape(NW, NCHUNK, CHUNK)
    ub = user_bias.reshape(-1)
    ib = item_bias.reshape(-1)
    mesh = plsc.VectorSubcoreMesh(core_axis_name="c", subcore_axis_name="s")
    k = functools.partial(
        pl.kernel,
        mesh=mesh,
        compiler_params=pltpu.CompilerParams(needs_layout_passes=False, use_tc_tiling_on_sc=False),
        out_type=jax.ShapeDtypeStruct((B,), jnp.float32),
        scratch_types=[
            pltpu.VMEM((NCHUNK, CHUNK), jnp.int32),
            pltpu.VMEM((NCHUNK, CHUNK), jnp.int32),
            pltpu.VMEM((BPW, D), jnp.float32),
            pltpu.VMEM((BPW, D), jnp.float32),
            pltpu.VMEM((BPW,), jnp.float32),
            pltpu.VMEM((BPW,), jnp.float32),
            pltpu.VMEM((BPW,), jnp.float32),
            pltpu.SemaphoreType.DMA,
        ],
    )(_body)
    return k(user_r, item_r, user_emb, item_emb, ub, ib)
